# Initial kernel scaffold; baseline (speedup 1.0000x reference)
import jax
import jax.numpy as jnp
from jax.experimental import pallas as pl


def kernel(title_ids, token_ids, title_table, text_table):
    raise NotImplementedError("write your pallas kernel here")


# speedup vs baseline: 8.8089x; 8.8089x over previous
import functools

import jax
import jax.numpy as jnp
from jax import lax
from jax.experimental import pallas as pl
from jax.experimental.pallas import tpu as pltpu
from jax.experimental.pallas import tpu_sc as plsc

NC = 2
NS = 16
NW = NC * NS

B = 16384
S = 20
D = 64
RPW = B // NW
CB = 32
NCHUNK = RPW // CB
TOK_PER_CHUNK = CB * S
IDX_ROWS = TOK_PER_CHUNK // 128


def _sc_body(title_ids_hbm, tokflat_hbm, title_tab_hbm, text_tab_hbm,
             out_hbm, idx_v, tidx_v, rows_v, tbuf_v, out_v, sem_g, sem_t):
    wid = lax.axis_index("s") * NC + lax.axis_index("c")

    def chunk_body(g, _):
        base = wid * RPW + g * CB
        pltpu.sync_copy(tokflat_hbm.at[pl.ds(base * S, TOK_PER_CHUNK)], idx_v)
        pltpu.sync_copy(title_ids_hbm.at[pl.ds(base, CB)], tidx_v)
        for i in range(TOK_PER_CHUNK // 16):
            t = idx_v[pl.ds(i * 16, 16)]
            idx_v[pl.ds(i * 16, 16)] = jnp.where(
                t == 0, jnp.int32(text_tab_hbm.shape[0] - 1), t)
        title_dma = pltpu.async_copy(title_tab_hbm.at[tidx_v], tbuf_v, sem_t)
        tok_dmas = [
            pltpu.async_copy(text_tab_hbm.at[idx_v.at[pl.ds(j * 128, 128)]],
                             rows_v.at[pl.ds(j * 128, 128)], sem_g)
            for j in range(IDX_ROWS)
        ]
        for dma in tok_dmas:
            dma.wait()
        title_dma.wait()

        def row_body(r, _):
            rb = r * S
            for d in range(4):
                acc = rows_v[rb, pl.ds(d * 16, 16)]
                for s in range(1, S):
                    acc = jnp.maximum(acc, rows_v[rb + s, pl.ds(d * 16, 16)])
                out_v[r, pl.ds(D + d * 16, 16)] = acc
                out_v[r, pl.ds(d * 16, 16)] = tbuf_v[r, pl.ds(d * 16, 16)]
            return _

        lax.fori_loop(0, CB, row_body, None)
        pltpu.sync_copy(out_v, out_hbm.at[pl.ds(base, CB)])
        return _

    lax.fori_loop(0, NCHUNK, chunk_body, None)


@jax.jit
def _run(title_ids, tokflat, title_table, text_aug):
    mesh = plsc.VectorSubcoreMesh(core_axis_name="c", subcore_axis_name="s")
    f = functools.partial(
        pl.kernel,
        out_type=jax.ShapeDtypeStruct((B, 2 * D), jnp.float32),
        mesh=mesh,
        scratch_types=[
            pltpu.VMEM((TOK_PER_CHUNK,), jnp.int32),
            pltpu.VMEM((CB,), jnp.int32),
            pltpu.VMEM((TOK_PER_CHUNK, 128), jnp.float32),
            pltpu.VMEM((CB, 128), jnp.float32),
            pltpu.VMEM((CB, 2 * D), jnp.float32),
            pltpu.SemaphoreType.DMA,
            pltpu.SemaphoreType.DMA,
        ],
    )(_sc_body)
    return f(title_ids, tokflat, title_table, text_aug)


def kernel(title_ids, token_ids, title_table, text_table):
    text_aug = jnp.concatenate(
        [text_table, jnp.full((1, D), -1e9, jnp.float32)], axis=0)
    text_aug = jnp.pad(text_aug, ((0, 0), (0, 128 - D)))
    title_pad = jnp.pad(title_table, ((0, 0), (0, 128 - D)))
    tokflat = token_ids.reshape(-1)
    return _run(title_ids, tokflat, title_pad, text_aug)
